# baseline (speedup 1.0000x reference)
import functools

import jax
import jax.numpy as jnp
from jax import lax
from jax.experimental import pallas as pl
from jax.experimental.pallas import tpu as pltpu
from jax.experimental.pallas import tpu_sc as plsc

T, B, V, L = 32, 32, 50000, 400
TB = T * B
LANES = 16
NC = 2
NBUF = 4
VSPLIT = 24960
HALVES = ((0, VSPLIT), (VSPLIT, V - VSPLIT))


def _dist_gen_body(vocab_hbm, attns_hbm, pg_hbm, src_hbm, out_hbm,
                   src_v, attn_v, pg_v, tmp_v, buf0, buf1, buf2, buf3,
                   in_sem0, in_sem1, in_sem2, in_sem3,
                   out_sem0, out_sem1, out_sem2, out_sem3):
    wid = lax.axis_index("s") * NC + lax.axis_index("c")
    bufs = (buf0, buf1, buf2, buf3)
    in_sems = (in_sem0, in_sem1, in_sem2, in_sem3)
    out_sems = (out_sem0, out_sem1, out_sem2, out_sem3)

    def refs(t, j):
        v0, w = HALVES[j & 1]
        r = (t + (j >> 1)) * B + wid
        src = vocab_hbm.at[r, pl.ds(v0, w)]
        dst = bufs[j].at[pl.ds(0, w)]
        out = out_hbm.at[r, pl.ds(v0, w)]
        return src, dst, out

    for j in range(NBUF):
        src, dst, _ = refs(0, j)
        pltpu.make_async_copy(src, dst, in_sems[j]).start()
    pltpu.sync_copy(src_hbm.at[wid], src_v)
    pltpu.sync_copy(pg_hbm.at[wid], pg_v)
    pltpu.sync_copy(attns_hbm.at[wid], attn_v)

    def step(g, carry):
        for j in range(NBUF):
            v0, w = HALVES[j & 1]
            t = g * 2 + (j >> 1)
            src, dst, out = refs(g * 2, j)
            pltpu.make_async_copy(src, dst, in_sems[j]).wait()

            tidx = jnp.zeros((LANES,), jnp.int32) + t
            p = plsc.load_gather(pg_v, [tidx])
            one_m_p = 1.0 - p

            for q in range(L // LANES):
                sl = pl.ds(q * LANES, LANES)
                s = src_v[sl] - v0
                m = (s >= 0) & (s < w)
                sc = jnp.where(m, s, 0)
                tmp_v[sl] = plsc.load_gather(bufs[j], [sc], mask=m)

            @plsc.parallel_loop(0, w // LANES, unroll=8)
            def scale_body(i):
                sl = pl.ds(i * LANES, LANES)
                bufs[j][sl] = bufs[j][sl] * p

            for q in range(L // LANES):
                sl = pl.ds(q * LANES, LANES)
                s = src_v[sl] - v0
                m = (s >= 0) & (s < w)
                sc = jnp.where(m, s, 0)
                a = attn_v[pl.ds(t * L + q * LANES, LANES)]
                val = one_m_p * a + p * tmp_v[sl]
                plsc.store_scatter(bufs[j], [sc], val, mask=m)

            pltpu.make_async_copy(dst, out, out_sems[j]).start()

        for j in range(NBUF):
            _, dst, out = refs(g * 2, j)
            pltpu.make_async_copy(dst, out, out_sems[j]).wait()

            @pl.when(g < T // 2 - 1)
            def _():
                src2, dst2, _ = refs(g * 2 + 2, j)
                pltpu.make_async_copy(src2, dst2, in_sems[j]).start()

        return carry

    lax.fori_loop(0, T // 2, step, 0)


@jax.jit
def _dist_gen(vocab_ds, attns_t, pg_bt, src_t):
    mesh = plsc.VectorSubcoreMesh(core_axis_name="c", subcore_axis_name="s")
    run = functools.partial(
        pl.kernel,
        out_type=jax.ShapeDtypeStruct((TB, V), jnp.float32),
        mesh=mesh,
        compiler_params=pltpu.CompilerParams(
            needs_layout_passes=False, use_tc_tiling_on_sc=True),
        scratch_types=[
            pltpu.VMEM((L,), jnp.int32),
            pltpu.VMEM((T * L,), jnp.float32),
            pltpu.VMEM((T,), jnp.float32),
            pltpu.VMEM((L,), jnp.float32),
            pltpu.VMEM((V - VSPLIT,), jnp.float32),
            pltpu.VMEM((V - VSPLIT,), jnp.float32),
            pltpu.VMEM((V - VSPLIT,), jnp.float32),
            pltpu.VMEM((V - VSPLIT,), jnp.float32),
        ] + [pltpu.SemaphoreType.DMA] * 8,
    )(_dist_gen_body)
    return run(vocab_ds, attns_t, pg_bt, src_t)


def kernel(vocab_ds, attns, p_gens, sources, decoder_batch_len):
    del decoder_batch_len
    pg_bt = p_gens.reshape(T, B).T.reshape(B, T)
    src_t = sources.T.reshape(B, L)
    attns_t = attns.reshape(T, B, L).transpose(1, 0, 2).reshape(B, T * L)
    return _dist_gen(vocab_ds, attns_t, pg_bt, src_t)
